# speedup vs baseline: 2.5935x; 2.5935x over previous
import functools

import jax
import jax.numpy as jnp
from jax import lax
from jax.experimental import pallas as pl
from jax.experimental.pallas import tpu as pltpu
from jax.experimental.pallas import tpu_sc as plsc

_NC = 2
_NS = 16
_NW = _NC * _NS
_CHUNK = 128


def _sc_gather_pair(ids_flat, elem_weight, hnet_weight):
    n = ids_flat.shape[0]
    emb = elem_weight.shape[1]
    per_w = n // _NW
    n_chunks = per_w // _CHUNK
    mesh = plsc.VectorSubcoreMesh(core_axis_name="c", subcore_axis_name="s")

    @functools.partial(
        pl.kernel,
        out_type=(
            jax.ShapeDtypeStruct((n, emb), jnp.float32),
            jax.ShapeDtypeStruct((n, emb), jnp.float32),
        ),
        mesh=mesh,
        scratch_types=[
            pltpu.VMEM((_CHUNK,), jnp.int32),
            pltpu.VMEM((_CHUNK, emb), jnp.float32),
            pltpu.VMEM((_CHUNK, emb), jnp.float32),
            pltpu.SemaphoreType.DMA,
            pltpu.SemaphoreType.DMA,
        ],
    )
    def sc_gather(ids_hbm, elem_hbm, hnet_hbm, out_e, out_h,
                  idx_v, erow_v, hrow_v, sem_e, sem_h):
        wid = lax.axis_index("s") * _NC + lax.axis_index("c")
        base = wid * per_w

        @pl.loop(0, n_chunks)
        def _(j):
            off = base + j * _CHUNK
            pltpu.sync_copy(ids_hbm.at[pl.ds(off, _CHUNK)], idx_v)
            cp_e = pltpu.async_copy(elem_hbm.at[idx_v], erow_v, sem_e)
            cp_h = pltpu.async_copy(hnet_hbm.at[idx_v], hrow_v, sem_h)
            cp_e.wait()
            cp_h.wait()
            pltpu.sync_copy(erow_v, out_e.at[pl.ds(off, _CHUNK)])
            pltpu.sync_copy(hrow_v, out_h.at[pl.ds(off, _CHUNK)])

    return sc_gather(ids_flat, elem_weight, hnet_weight)


def _tc_combine(hnet_flat, erow, hrow, lin_weight, blk=2048):
    n, nhp = hnet_flat.shape
    emb = lin_weight.shape[0]

    def body(hnet_ref, e_ref, h_ref, lin_ref, out_ref):
        scal = lax.dot_general(
            hnet_ref[...], lin_ref[...],
            (((1,), (1,)), ((), ())),
            preferred_element_type=jnp.float32,
        )
        out_ref[...] = e_ref[...] + h_ref[...] * scal

    return pl.pallas_call(
        body,
        grid=(n // blk,),
        in_specs=[
            pl.BlockSpec((blk, nhp), lambda i: (i, 0)),
            pl.BlockSpec((blk, emb), lambda i: (i, 0)),
            pl.BlockSpec((blk, emb), lambda i: (i, 0)),
            pl.BlockSpec((emb, nhp), lambda i: (0, 0)),
        ],
        out_specs=pl.BlockSpec((blk, emb), lambda i: (i, 0)),
        out_shape=jax.ShapeDtypeStruct((n, emb), jnp.float32),
    )(hnet_flat, erow, hrow, lin_weight)


def kernel(input_ids, hnet_tensor, elem_weight, hnet_weight, lin_weight):
    b, l = input_ids.shape
    n = b * l
    emb = elem_weight.shape[1]
    ids_flat = input_ids.reshape(n).astype(jnp.int32)
    erow, hrow = _sc_gather_pair(ids_flat, elem_weight, hnet_weight)
    hnet_flat = hnet_tensor.reshape(n, hnet_tensor.shape[2])
    out = _tc_combine(hnet_flat, erow, hrow, lin_weight)
    return out.reshape(b, l, emb)
